# baseline (speedup 1.0000x reference)
import jax
import jax.numpy as jnp
from jax.experimental import pallas as pl

PRE_NMS_THRESH = 0.3
K_TOP = 1000
DOWNSAMPLE = 32.0

N, C, T = 32, 4, 20000


def _mask_body(cls_ref, iou_ref, out_ref):
    a = cls_ref[...]
    b = iou_ref[...]
    sa = 1.0 / (1.0 + jnp.exp(-a))
    sb = 1.0 / (1.0 + jnp.exp(-b))
    s = sa * sb
    out_ref[...] = jnp.where(s > PRE_NMS_THRESH, s, 0.0)


def _decode_body(tv_ref, ti_ref, gl_ref, g0_ref, g1_ref,
                 d0_ref, d1_ref, sc_ref, nl_ref, lb_ref):
    tv = tv_ref[...]
    ti = ti_ref[...]
    gl = gl_ref[...]
    g0 = g0_ref[...]
    g1 = g1_ref[...]
    start = jnp.clip((gl - g0) / DOWNSAMPLE, 0.0, 1.0)
    end = jnp.clip((gl + g1) / DOWNSAMPLE, 0.0, 1.0)
    valid = (tv > PRE_NMS_THRESH) & ((end - start) >= 0.0)
    vf = valid.astype(jnp.float32)
    safe = jnp.where(valid, tv, 1.0)
    d0_ref[...] = start * vf
    d1_ref[...] = end * vf
    sc_ref[...] = jnp.sqrt(safe) * vf
    nl_ref[...] = (gl / DOWNSAMPLE) * vf
    lb_ref[...] = (ti % C) + 1


def kernel(locations, box_cls, box_regression, iou_scores):
    f32 = jnp.float32
    i32 = jnp.int32
    BN = 8
    m = pl.pallas_call(
        _mask_body,
        grid=(N // BN,),
        in_specs=[
            pl.BlockSpec((BN, C, T), lambda g: (g, 0, 0)),
            pl.BlockSpec((BN, C, T), lambda g: (g, 0, 0)),
        ],
        out_specs=pl.BlockSpec((BN, C, T), lambda g: (g, 0, 0)),
        out_shape=jax.ShapeDtypeStruct((N, C, T), jnp.float32),
    )(box_cls, iou_scores)

    flat = jnp.transpose(m, (0, 2, 1)).reshape(N, T * C)
    tv, ti = jax.lax.top_k(flat, K_TOP)

    box_loc = ti // C
    gl = locations[box_loc]
    g0 = jnp.take_along_axis(box_regression[:, 0, :], box_loc, axis=1)
    g1 = jnp.take_along_axis(box_regression[:, 1, :], box_loc, axis=1)

    spec = pl.BlockSpec((N, K_TOP), lambda: (0, 0))
    d0, d1, scores, nl, lb = pl.pallas_call(
        _decode_body,
        in_specs=[spec] * 5,
        out_specs=[spec] * 5,
        out_shape=[
            jax.ShapeDtypeStruct((N, K_TOP), f32),
            jax.ShapeDtypeStruct((N, K_TOP), f32),
            jax.ShapeDtypeStruct((N, K_TOP), f32),
            jax.ShapeDtypeStruct((N, K_TOP), f32),
            jax.ShapeDtypeStruct((N, K_TOP), i32),
        ],
    )(tv, ti, gl, g0, g1)

    detections = jnp.stack([d0, d1], axis=-1)
    return (detections, scores, nl, lb)


# speedup vs baseline: 1.0053x; 1.0053x over previous
import jax
import jax.numpy as jnp
from jax.experimental import pallas as pl

PRE_NMS_THRESH = 0.3
K_TOP = 1000
DOWNSAMPLE = 32.0

N, C, T = 32, 4, 20000
KPAD = 1024


def _mask_body(cls_ref, iou_ref, out_ref):
    a = cls_ref[...]
    b = iou_ref[...]
    sa = 1.0 / (1.0 + jnp.exp(-a))
    sb = 1.0 / (1.0 + jnp.exp(-b))
    s = sa * sb
    out_ref[...] = jnp.where(s > PRE_NMS_THRESH, s, 0.0)


def _decode_body(tv_ref, ti_ref, gl_ref, g0_ref, g1_ref,
                 d0_ref, d1_ref, sc_ref, nl_ref, lb_ref):
    tv = tv_ref[...]
    ti = ti_ref[...]
    gl = gl_ref[...]
    g0 = g0_ref[...]
    g1 = g1_ref[...]
    start = jnp.clip((gl - g0) / DOWNSAMPLE, 0.0, 1.0)
    end = jnp.clip((gl + g1) / DOWNSAMPLE, 0.0, 1.0)
    valid = (tv > PRE_NMS_THRESH) & ((end - start) >= 0.0)
    vf = valid.astype(jnp.float32)
    safe = jnp.where(valid, tv, 1.0)
    d0_ref[...] = start * vf
    d1_ref[...] = end * vf
    sc_ref[...] = jnp.sqrt(safe) * vf
    nl_ref[...] = (gl / DOWNSAMPLE) * vf
    lb_ref[...] = (ti & (C - 1)) + 1


def kernel(locations, box_cls, box_regression, iou_scores):
    f32 = jnp.float32
    i32 = jnp.int32
    BN = 8
    m = pl.pallas_call(
        _mask_body,
        grid=(N // BN,),
        in_specs=[
            pl.BlockSpec((BN, C, T), lambda g: (g, 0, 0)),
            pl.BlockSpec((BN, C, T), lambda g: (g, 0, 0)),
        ],
        out_specs=pl.BlockSpec((BN, C, T), lambda g: (g, 0, 0)),
        out_shape=jax.ShapeDtypeStruct((N, C, T), f32),
    )(box_cls, iou_scores)

    mt = jnp.transpose(m, (0, 2, 1)).reshape(N, T * C)
    topv, topi = jax.lax.top_k(mt, K_TOP)

    box_loc = topi // C
    gl = locations[box_loc]
    reg = jnp.transpose(box_regression, (0, 2, 1))
    per_reg = jnp.take_along_axis(reg, box_loc[..., None], axis=1)
    g0 = per_reg[..., 0]
    g1 = per_reg[..., 1]

    pad = ((0, 0), (0, KPAD - K_TOP))
    tvp = jnp.pad(topv, pad)
    tip = jnp.pad(topi, pad)
    glp = jnp.pad(gl, pad)
    g0p = jnp.pad(g0, pad)
    g1p = jnp.pad(g1, pad)

    spec = pl.BlockSpec((N, KPAD), lambda: (0, 0))
    d0, d1, scores, nl, lb = pl.pallas_call(
        _decode_body,
        in_specs=[spec] * 5,
        out_specs=[spec] * 5,
        out_shape=[
            jax.ShapeDtypeStruct((N, KPAD), f32),
            jax.ShapeDtypeStruct((N, KPAD), f32),
            jax.ShapeDtypeStruct((N, KPAD), f32),
            jax.ShapeDtypeStruct((N, KPAD), f32),
            jax.ShapeDtypeStruct((N, KPAD), i32),
        ],
    )(tvp, tip, glp, g0p, g1p)

    detections = jnp.stack([d0[:, :K_TOP], d1[:, :K_TOP]], axis=-1)
    return (detections, scores[:, :K_TOP], nl[:, :K_TOP], lb[:, :K_TOP])
